# baseline (speedup 1.0000x reference)
import jax
import jax.numpy as jnp
from jax.experimental import pallas as pl
from jax.experimental.pallas import tpu as pltpu

_ISSUE_UNROLL = 16
_LANES = 128


def _fused_kernel(Sw, Sp, ids_smem, w_hbm, pos_hbm, out_ref, sem_w, sem_p):
    tile = out_ref.shape[0]
    base = pl.program_id(0) * tile

    pcopy = pltpu.make_async_copy(
        pos_hbm.at[pl.ds(base, tile)],
        out_ref.at[:, pl.ds(Sw, Sp)],
        sem_p,
    )
    pcopy.start()

    unroll = _ISSUE_UNROLL if tile % _ISSUE_UNROLL == 0 else 1

    @pl.loop(0, tile // unroll)
    def _issue(c):
        r0 = c * unroll
        for u in range(unroll):
            r = r0 + u
            tok = ids_smem[base + r]
            pltpu.make_async_copy(
                w_hbm.at[tok],
                out_ref.at[r, pl.ds(0, Sw)],
                sem_w,
            ).start()

    pltpu.make_async_copy(
        w_hbm.at[pl.ds(0, tile)],
        out_ref.at[:, pl.ds(0, Sw)],
        sem_w,
    ).wait()
    pcopy.wait()


def _pick_tile(L):
    if L <= 1024:
        return L
    for t in (1024, 512, 256, 128, 64, 32, 16, 8):
        if L % t == 0:
            return t
    return L


def kernel(ids, word_table, pos_table):
    L = ids.shape[0]
    V, Dw = word_table.shape
    P, Dp = pos_table.shape
    assert P >= L, "position table must cover the sequence length"
    assert Dw % _LANES == 0 and Dp % _LANES == 0
    Sw, Sp = Dw // _LANES, Dp // _LANES

    ids = ids.astype(jnp.int32)
    w3 = word_table.reshape(V, Sw, _LANES)
    pos3 = pos_table[:L].reshape(L, Sp, _LANES)
    tile = _pick_tile(L)
    grid = (L // tile,)

    out = pl.pallas_call(
        lambda *refs: _fused_kernel(Sw, Sp, *refs),
        out_shape=jax.ShapeDtypeStruct((L, Sw + Sp, _LANES), word_table.dtype),
        grid_spec=pltpu.PrefetchScalarGridSpec(
            num_scalar_prefetch=1,
            grid=grid,
            in_specs=[
                pl.BlockSpec(memory_space=pl.ANY),
                pl.BlockSpec(memory_space=pl.ANY),
            ],
            out_specs=pl.BlockSpec((tile, Sw + Sp, _LANES),
                                   lambda i, ids_ref: (i, 0, 0)),
            scratch_shapes=[pltpu.SemaphoreType.DMA(()),
                            pltpu.SemaphoreType.DMA(())],
        ),
        compiler_params=pltpu.CompilerParams(
            dimension_semantics=("parallel",),
            disable_bounds_checks=True,
        ),
    )(ids, w3, pos3)
    return out.reshape(L, Dw + Dp)


# speedup vs baseline: 5.9621x; 5.9621x over previous
import jax
import jax.numpy as jnp
from jax.experimental import pallas as pl
from jax.experimental.pallas import tpu as pltpu

_ISSUE_UNROLL = 16
_CHUNK = 256


def _pipelined_kernel(Dw, Dp, rows, chunk,
                      ids_smem, w_hbm, pos_hbm, out_hbm,
                      buf0, buf1, sem_w0, sem_w1, sem_p0, sem_p1,
                      sem_o0, sem_o1):
    base = pl.program_id(0) * rows
    nchunks = rows // chunk
    bufs = (buf0, buf1)
    sems_w = (sem_w0, sem_w1)
    sems_p = (sem_p0, sem_p1)
    sems_o = (sem_o0, sem_o1)

    unroll = _ISSUE_UNROLL if chunk % _ISSUE_UNROLL == 0 else 1

    def issue_chunk(c, buf, sw, sp):
        start = base + c * chunk
        pltpu.make_async_copy(
            pos_hbm.at[pl.ds(start, chunk)],
            buf.at[:, pl.ds(Dw, Dp)],
            sp,
        ).start()

        @pl.loop(0, chunk // unroll)
        def _issue(cc):
            r0 = cc * unroll
            for u in range(unroll):
                r = r0 + u
                tok = ids_smem[start + r]
                pltpu.make_async_copy(
                    w_hbm.at[tok],
                    buf.at[r, pl.ds(0, Dw)],
                    sw,
                ).start()

    def wait_chunk(buf, sw, sp):
        pltpu.make_async_copy(
            w_hbm.at[pl.ds(0, chunk)],
            buf.at[:, pl.ds(0, Dw)],
            sw,
        ).wait()
        pltpu.make_async_copy(
            pos_hbm.at[pl.ds(0, chunk)],
            buf.at[:, pl.ds(Dw, Dp)],
            sp,
        ).wait()

    def start_out(c, buf, so):
        pltpu.make_async_copy(
            buf,
            out_hbm.at[pl.ds(base + c * chunk, chunk)],
            so,
        ).start()

    def wait_out(so):
        pltpu.make_async_copy(
            buf0,
            out_hbm.at[pl.ds(base, chunk)],
            so,
        ).wait()

    for c in range(nchunks):
        b = c & 1
        if c >= 2:
            wait_out(sems_o[b])
        issue_chunk(c, bufs[b], sems_w[b], sems_p[b])
        if c >= 1:
            p = 1 - b
            wait_chunk(bufs[p], sems_w[p], sems_p[p])
            start_out(c - 1, bufs[p], sems_o[p])

    last = nchunks - 1
    b = last & 1
    wait_chunk(bufs[b], sems_w[b], sems_p[b])
    start_out(last, bufs[b], sems_o[b])
    if nchunks >= 2:
        wait_out(sems_o[1 - b])
    wait_out(sems_o[b])


def _pick_chunk(rows):
    for c in (_CHUNK, 128, 64, 32, 16, 8):
        if rows % c == 0:
            return c
    return rows


def kernel(ids, word_table, pos_table):
    L = ids.shape[0]
    V, Dw = word_table.shape
    P, Dp = pos_table.shape
    assert P >= L, "position table must cover the sequence length"

    ids = ids.astype(jnp.int32)
    pos_used = pos_table[:L]
    ncores = 2 if L % 2 == 0 else 1
    rows = L // ncores
    chunk = _pick_chunk(rows)

    out = pl.pallas_call(
        lambda *refs: _pipelined_kernel(Dw, Dp, rows, chunk, *refs),
        out_shape=jax.ShapeDtypeStruct((L, Dw + Dp), word_table.dtype),
        grid_spec=pltpu.PrefetchScalarGridSpec(
            num_scalar_prefetch=1,
            grid=(ncores,),
            in_specs=[
                pl.BlockSpec(memory_space=pl.ANY),
                pl.BlockSpec(memory_space=pl.ANY),
            ],
            out_specs=pl.BlockSpec(memory_space=pl.ANY),
            scratch_shapes=[
                pltpu.VMEM((chunk, Dw + Dp), word_table.dtype),
                pltpu.VMEM((chunk, Dw + Dp), word_table.dtype),
                pltpu.SemaphoreType.DMA(()),
                pltpu.SemaphoreType.DMA(()),
                pltpu.SemaphoreType.DMA(()),
                pltpu.SemaphoreType.DMA(()),
                pltpu.SemaphoreType.DMA(()),
                pltpu.SemaphoreType.DMA(()),
            ],
        ),
        compiler_params=pltpu.CompilerParams(
            dimension_semantics=("parallel",),
            disable_bounds_checks=True,
        ),
    )(ids, word_table, pos_used)
    return out


# speedup vs baseline: 7.4712x; 1.2531x over previous
import jax
import jax.numpy as jnp
from jax.experimental import pallas as pl
from jax.experimental.pallas import tpu as pltpu

_ISSUE_UNROLL = 32
_TILE = 1024


def _fused_kernel(Dw, Dp, ids_smem, w_hbm, pos_hbm, out_ref, sem_w, sem_p):
    tile = out_ref.shape[0]
    base = pl.program_id(0) * tile

    pcopy = pltpu.make_async_copy(
        pos_hbm.at[pl.ds(base, tile)],
        out_ref.at[:, pl.ds(Dw, Dp)],
        sem_p,
    )
    pcopy.start()

    unroll = _ISSUE_UNROLL if tile % _ISSUE_UNROLL == 0 else 1

    @pl.loop(0, tile // unroll)
    def _issue(c):
        r0 = c * unroll
        for u in range(unroll):
            r = r0 + u
            tok = ids_smem[base + r]
            pltpu.make_async_copy(
                w_hbm.at[tok],
                out_ref.at[r, pl.ds(0, Dw)],
                sem_w,
            ).start()

    pltpu.make_async_copy(
        w_hbm.at[pl.ds(0, tile)],
        out_ref.at[:, pl.ds(0, Dw)],
        sem_w,
    ).wait()
    pcopy.wait()


def _pick_tile(L):
    if L <= _TILE:
        return L
    for t in (_TILE, 512, 256, 128, 64, 32, 16, 8):
        if L % t == 0:
            return t
    return L


def kernel(ids, word_table, pos_table):
    L = ids.shape[0]
    V, Dw = word_table.shape
    P, Dp = pos_table.shape
    assert P >= L, "position table must cover the sequence length"

    ids = ids.astype(jnp.int32)
    pos_used = pos_table[:L]
    tile = _pick_tile(L)
    grid = (L // tile,)

    out = pl.pallas_call(
        lambda *refs: _fused_kernel(Dw, Dp, *refs),
        out_shape=jax.ShapeDtypeStruct((L, Dw + Dp), word_table.dtype),
        grid_spec=pltpu.PrefetchScalarGridSpec(
            num_scalar_prefetch=1,
            grid=grid,
            in_specs=[
                pl.BlockSpec(memory_space=pl.ANY),
                pl.BlockSpec(memory_space=pl.ANY),
            ],
            out_specs=pl.BlockSpec((tile, Dw + Dp), lambda i, ids_ref: (i, 0)),
            scratch_shapes=[pltpu.SemaphoreType.DMA(()),
                            pltpu.SemaphoreType.DMA(())],
        ),
        compiler_params=pltpu.CompilerParams(
            dimension_semantics=("parallel",),
            disable_bounds_checks=True,
        ),
    )(ids, word_table, pos_used)
    return out
